# Initial kernel scaffold; baseline (speedup 1.0000x reference)
import jax
import jax.numpy as jnp
from jax.experimental import pallas as pl


def kernel(pred_cls, pred_loc, anchor):
    raise NotImplementedError("write your pallas kernel here")


# speedup vs baseline: 30.4534x; 30.4534x over previous
import jax
import jax.numpy as jnp
from jax.experimental import pallas as pl
from jax.experimental.pallas import tpu as pltpu

_TOPK = 6000
_OUT_K = 300
_IOU = 0.7
_B = 256
_N_PAD = 6144
_NB = _N_PAD // _B


def _nms_body(loc_ref, anc_ref, box_ref, keep_ref, area_ref, ms_ref):
    loc = loc_ref[...]
    anc = anc_ref[...]

    cx = loc[0:1] * 0.1 * anc[2:3] + anc[0:1]
    cy = loc[1:2] * 0.1 * anc[3:4] + anc[1:2]
    w = jnp.exp(loc[2:3] * 0.2) * anc[2:3]
    h = jnp.exp(loc[3:4] * 0.2) * anc[3:4]
    x0 = jnp.clip(cx - w / 2.0, 0.0, 1.0)
    y0 = jnp.clip(cy - h / 2.0, 0.0, 1.0)
    x1 = jnp.clip(cx + w / 2.0, 0.0, 1.0)
    y1 = jnp.clip(cy + h / 2.0, 0.0, 1.0)
    box_ref[...] = jnp.concatenate([x0, y0, x1, y1], axis=0)
    area_ref[...] = (x1 - x0) * (y1 - y0)
    keep_ref[...] = jnp.ones((1, _N_PAD), jnp.float32)

    lane = jax.lax.broadcasted_iota(jnp.int32, (1, _B), 1)
    row_i = jax.lax.broadcasted_iota(jnp.int32, (_B, _B), 0)
    col_j = jax.lax.broadcasted_iota(jnp.int32, (_B, _B), 1)
    tri = col_j > row_i

    def outer(k, carry):
        s = k * _B
        blk = box_ref[:, pl.ds(s, _B)]
        bx0, by0, bx1, by1 = blk[0:1], blk[1:2], blk[2:3], blk[3:4]
        bar = area_ref[0:1, pl.ds(s, _B)]
        bx0c = bx0.reshape(_B, 1)
        by0c = by0.reshape(_B, 1)
        bx1c = bx1.reshape(_B, 1)
        by1c = by1.reshape(_B, 1)
        barc = bar.reshape(_B, 1)

        iw = jnp.clip(jnp.minimum(bx1c, bx1) - jnp.maximum(bx0c, bx0), 0.0, None)
        ih = jnp.clip(jnp.minimum(by1c, by1) - jnp.maximum(by0c, by0), 0.0, None)
        inter = iw * ih
        iou = inter / jnp.maximum(bar + barc - inter, 1e-12)
        ms_ref[...] = jnp.where(tri & (iou > _IOU), 1.0, 0.0)

        kb = keep_ref[0:1, pl.ds(s, _B)]

        def inner(i, kb):
            ki = jnp.max(jnp.where(lane == i, kb, 0.0))
            row = ms_ref[pl.ds(i, 1), :]
            return kb * (1.0 - row * ki)

        kb = jax.lax.fori_loop(0, _B, inner, kb)
        keep_ref[0:1, pl.ds(s, _B)] = kb
        kbc = kb.reshape(_B, 1)

        def cross(m, carry):
            t = m * _B
            mblk = box_ref[:, pl.ds(t, _B)]
            mar = area_ref[0:1, pl.ds(t, _B)]
            iw = jnp.clip(jnp.minimum(bx1c, mblk[2:3]) - jnp.maximum(bx0c, mblk[0:1]), 0.0, None)
            ih = jnp.clip(jnp.minimum(by1c, mblk[3:4]) - jnp.maximum(by0c, mblk[1:2]), 0.0, None)
            inter = iw * ih
            iou = inter / jnp.maximum(mar + barc - inter, 1e-12)
            sup = jnp.where(iou > _IOU, 1.0, 0.0) * kbc
            supc = jnp.max(sup, axis=0, keepdims=True)
            keep_ref[0:1, pl.ds(t, _B)] = keep_ref[0:1, pl.ds(t, _B)] * (1.0 - supc)
            return carry

        return jax.lax.fori_loop(k + 1, _NB, cross, carry)

    jax.lax.fori_loop(0, _NB, outer, 0)


_nms = pl.pallas_call(
    _nms_body,
    out_shape=[
        jax.ShapeDtypeStruct((4, _N_PAD), jnp.float32),
        jax.ShapeDtypeStruct((1, _N_PAD), jnp.float32),
    ],
    scratch_shapes=[
        pltpu.VMEM((1, _N_PAD), jnp.float32),
        pltpu.VMEM((_B, _B), jnp.float32),
    ],
)


@jax.jit
def kernel(pred_cls, pred_loc, anchor):
    pc = jnp.transpose(pred_cls, (0, 2, 3, 1)).reshape(-1, 2)
    obj = jax.nn.sigmoid(pc[:, 1])
    ploc = jnp.transpose(pred_loc, (0, 2, 3, 1)).reshape(-1, 4)
    _, top = jax.lax.top_k(obj, _TOPK)
    loc_top = jnp.take(ploc, top, axis=0)
    anc_top = jnp.take(anchor, top, axis=0)
    pad = ((0, _N_PAD - _TOPK), (0, 0))
    boxes_t, keep = _nms(jnp.pad(loc_top, pad).T, jnp.pad(anc_top, pad).T)
    keepb = keep[0, :_TOPK] > 0.5
    kidx = jnp.nonzero(keepb, size=_OUT_K, fill_value=-1)[0]
    valid = kidx >= 0
    gathered = boxes_t.T[jnp.clip(kidx, 0, _TOPK - 1)]
    return jnp.where(valid[:, None], gathered, 0.0)


# speedup vs baseline: 83.2255x; 2.7329x over previous
import jax
import jax.numpy as jnp
from jax.experimental import pallas as pl
from jax.experimental.pallas import tpu as pltpu

_TOPK = 6000
_OUT_K = 300
_IOU = 0.7
_B = 256
_N_PAD = 6144
_NB = _N_PAD // _B


def _nms_body(loc_ref, anc_ref, box_ref, keep_ref, area_ref):
    loc = loc_ref[...]
    anc = anc_ref[...]

    cx = loc[0:1] * 0.1 * anc[2:3] + anc[0:1]
    cy = loc[1:2] * 0.1 * anc[3:4] + anc[1:2]
    w = jnp.exp(loc[2:3] * 0.2) * anc[2:3]
    h = jnp.exp(loc[3:4] * 0.2) * anc[3:4]
    x0 = jnp.clip(cx - w / 2.0, 0.0, 1.0)
    y0 = jnp.clip(cy - h / 2.0, 0.0, 1.0)
    x1 = jnp.clip(cx + w / 2.0, 0.0, 1.0)
    y1 = jnp.clip(cy + h / 2.0, 0.0, 1.0)
    box_ref[...] = jnp.concatenate([x0, y0, x1, y1], axis=0)
    area_ref[...] = (x1 - x0) * (y1 - y0)
    keep_ref[...] = jnp.ones((1, _N_PAD), jnp.float32)

    row_i = jax.lax.broadcasted_iota(jnp.int32, (_B, _B), 0)
    col_j = jax.lax.broadcasted_iota(jnp.int32, (_B, _B), 1)
    tri = col_j > row_i

    def outer(k, carry):
        s = k * _B
        blk = box_ref[:, pl.ds(s, _B)]
        bx0, by0, bx1, by1 = blk[0:1], blk[1:2], blk[2:3], blk[3:4]
        bar = area_ref[0:1, pl.ds(s, _B)]
        bx0c = bx0.reshape(_B, 1)
        by0c = by0.reshape(_B, 1)
        bx1c = bx1.reshape(_B, 1)
        by1c = by1.reshape(_B, 1)
        barc = bar.reshape(_B, 1)

        iw = jnp.clip(jnp.minimum(bx1c, bx1) - jnp.maximum(bx0c, bx0), 0.0, None)
        ih = jnp.clip(jnp.minimum(by1c, by1) - jnp.maximum(by0c, by0), 0.0, None)
        inter = iw * ih
        iou = inter / jnp.maximum(bar + barc - inter, 1e-12)
        ms = jnp.where(tri & (iou > _IOU), 1.0, 0.0)

        init_kb = keep_ref[0:1, pl.ds(s, _B)]

        def jac_cond(state):
            return state[1]

        def jac_body(state):
            kb, _ = state
            sup = jnp.max(ms * kb.reshape(_B, 1), axis=0, keepdims=True)
            new = init_kb * (1.0 - sup)
            return new, jnp.any(new != kb)

        kb, _ = jax.lax.while_loop(jac_cond, jac_body, (init_kb, True))
        keep_ref[0:1, pl.ds(s, _B)] = kb
        kbc = kb.reshape(_B, 1)

        def cross(m, carry):
            t = m * _B
            mblk = box_ref[:, pl.ds(t, _B)]
            mar = area_ref[0:1, pl.ds(t, _B)]
            iw = jnp.clip(jnp.minimum(bx1c, mblk[2:3]) - jnp.maximum(bx0c, mblk[0:1]), 0.0, None)
            ih = jnp.clip(jnp.minimum(by1c, mblk[3:4]) - jnp.maximum(by0c, mblk[1:2]), 0.0, None)
            inter = iw * ih
            iou = inter / jnp.maximum(mar + barc - inter, 1e-12)
            sup = jnp.where(iou > _IOU, 1.0, 0.0) * kbc
            supc = jnp.max(sup, axis=0, keepdims=True)
            keep_ref[0:1, pl.ds(t, _B)] = keep_ref[0:1, pl.ds(t, _B)] * (1.0 - supc)
            return carry

        return jax.lax.fori_loop(k + 1, _NB, cross, carry)

    jax.lax.fori_loop(0, _NB, outer, 0)


_nms = pl.pallas_call(
    _nms_body,
    out_shape=[
        jax.ShapeDtypeStruct((4, _N_PAD), jnp.float32),
        jax.ShapeDtypeStruct((1, _N_PAD), jnp.float32),
    ],
    scratch_shapes=[
        pltpu.VMEM((1, _N_PAD), jnp.float32),
    ],
)


@jax.jit
def kernel(pred_cls, pred_loc, anchor):
    pc = jnp.transpose(pred_cls, (0, 2, 3, 1)).reshape(-1, 2)
    obj = jax.nn.sigmoid(pc[:, 1])
    ploc = jnp.transpose(pred_loc, (0, 2, 3, 1)).reshape(-1, 4)
    _, top = jax.lax.top_k(obj, _TOPK)
    loc_top = jnp.take(ploc, top, axis=0)
    anc_top = jnp.take(anchor, top, axis=0)
    pad = ((0, _N_PAD - _TOPK), (0, 0))
    boxes_t, keep = _nms(jnp.pad(loc_top, pad).T, jnp.pad(anc_top, pad).T)
    keepb = keep[0, :_TOPK] > 0.5
    kidx = jnp.nonzero(keepb, size=_OUT_K, fill_value=-1)[0]
    valid = kidx >= 0
    gathered = boxes_t.T[jnp.clip(kidx, 0, _TOPK - 1)]
    return jnp.where(valid[:, None], gathered, 0.0)


# speedup vs baseline: 112.9946x; 1.3577x over previous
import jax
import jax.numpy as jnp
from jax.experimental import pallas as pl
from jax.experimental.pallas import tpu as pltpu

_TOPK = 6000
_OUT_K = 300
_IOU = 0.7
_B = 256
_N_PAD = 6144
_NB = _N_PAD // _B


def _nms_body(loc_ref, anc_ref, box_ref, keep_ref, area_ref):
    loc = loc_ref[...]
    anc = anc_ref[...]

    cx = loc[0:1] * 0.1 * anc[2:3] + anc[0:1]
    cy = loc[1:2] * 0.1 * anc[3:4] + anc[1:2]
    w = jnp.exp(loc[2:3] * 0.2) * anc[2:3]
    h = jnp.exp(loc[3:4] * 0.2) * anc[3:4]
    x0 = jnp.clip(cx - w / 2.0, 0.0, 1.0)
    y0 = jnp.clip(cy - h / 2.0, 0.0, 1.0)
    x1 = jnp.clip(cx + w / 2.0, 0.0, 1.0)
    y1 = jnp.clip(cy + h / 2.0, 0.0, 1.0)
    box_ref[...] = jnp.concatenate([x0, y0, x1, y1], axis=0)
    area_ref[...] = (x1 - x0) * (y1 - y0)
    keep_ref[...] = jnp.ones((1, _N_PAD), jnp.float32)

    row_i = jax.lax.broadcasted_iota(jnp.int32, (_B, _B), 0)
    col_j = jax.lax.broadcasted_iota(jnp.int32, (_B, _B), 1)
    tri = col_j > row_i

    def outer(k, carry):
        s = k * _B
        blk = box_ref[:, pl.ds(s, _B)]
        bx0, by0, bx1, by1 = blk[0:1], blk[1:2], blk[2:3], blk[3:4]
        bar = area_ref[0:1, pl.ds(s, _B)]
        bx0c = bx0.reshape(_B, 1)
        by0c = by0.reshape(_B, 1)
        bx1c = bx1.reshape(_B, 1)
        by1c = by1.reshape(_B, 1)
        barc = bar.reshape(_B, 1)

        iw = jnp.clip(jnp.minimum(bx1c, bx1) - jnp.maximum(bx0c, bx0), 0.0, None)
        ih = jnp.clip(jnp.minimum(by1c, by1) - jnp.maximum(by0c, by0), 0.0, None)
        inter = iw * ih
        iou = inter / jnp.maximum(bar + barc - inter, 1e-12)
        ms = jnp.where(tri & (iou > _IOU), 1.0, 0.0)

        init_kb = keep_ref[0:1, pl.ds(s, _B)]

        def jac_cond(state):
            return state[1]

        def jac_body(state):
            kb, _ = state
            sup = jnp.max(ms * kb.reshape(_B, 1), axis=0, keepdims=True)
            new = init_kb * (1.0 - sup)
            return new, jnp.any(new != kb)

        kb, _ = jax.lax.while_loop(jac_cond, jac_body, (init_kb, True))
        keep_ref[0:1, pl.ds(s, _B)] = kb
        kbc = kb.reshape(_B, 1)

        def cross(m, carry):
            t = m * _B
            mblk = box_ref[:, pl.ds(t, _B)]
            mar = area_ref[0:1, pl.ds(t, _B)]
            iw = jnp.clip(jnp.minimum(bx1c, mblk[2:3]) - jnp.maximum(bx0c, mblk[0:1]), 0.0, None)
            ih = jnp.clip(jnp.minimum(by1c, mblk[3:4]) - jnp.maximum(by0c, mblk[1:2]), 0.0, None)
            inter = iw * ih
            iou = inter / jnp.maximum(mar + barc - inter, 1e-12)
            sup = jnp.where(iou > _IOU, 1.0, 0.0) * kbc
            supc = jnp.max(sup, axis=0, keepdims=True)
            keep_ref[0:1, pl.ds(t, _B)] = keep_ref[0:1, pl.ds(t, _B)] * (1.0 - supc)
            return carry

        return jax.lax.fori_loop(k + 1, _NB, cross, carry)

    jax.lax.fori_loop(0, _NB, outer, 0)


_N_OBJ = 90000
_N_OBJ_PAD = 90112


def _bisect_body(obj_ref, thr_ref):
    bits = jax.lax.bitcast_convert_type(obj_ref[...], jnp.int32)

    def body(it, lohi):
        lo, hi = lohi
        mid = (lo + hi) // 2
        cnt = jnp.sum(jnp.where(bits >= mid, 1.0, 0.0))
        big = cnt >= _TOPK
        return jnp.where(big, mid, lo), jnp.where(big, hi, mid)

    lo, _ = jax.lax.fori_loop(
        0, 31, body, (jnp.int32(0), jnp.int32(0x3F800001))
    )
    thr_ref[...] = jnp.zeros((1, 128), jnp.int32) + lo


_bisect = pl.pallas_call(
    _bisect_body,
    out_shape=jax.ShapeDtypeStruct((1, 128), jnp.int32),
)


_nms = pl.pallas_call(
    _nms_body,
    out_shape=[
        jax.ShapeDtypeStruct((4, _N_PAD), jnp.float32),
        jax.ShapeDtypeStruct((1, _N_PAD), jnp.float32),
    ],
    scratch_shapes=[
        pltpu.VMEM((1, _N_PAD), jnp.float32),
    ],
)


@jax.jit
def kernel(pred_cls, pred_loc, anchor):
    pc = jnp.transpose(pred_cls, (0, 2, 3, 1)).reshape(-1, 2)
    obj = jax.nn.sigmoid(pc[:, 1])
    ploc = jnp.transpose(pred_loc, (0, 2, 3, 1)).reshape(-1, 4)
    objp = jnp.pad(obj, (0, _N_OBJ_PAD - _N_OBJ), constant_values=-1.0)
    thr_bits = _bisect(objp.reshape(704, 128))[0, 0]
    thr = jax.lax.bitcast_convert_type(thr_bits, jnp.float32)
    gt = obj > thr
    eq = obj == thr
    k_eq = _TOPK - jnp.sum(gt.astype(jnp.int32))
    sel = gt | (eq & (jnp.cumsum(eq.astype(jnp.int32)) <= k_eq))
    idxs = jnp.nonzero(sel, size=_TOPK, fill_value=0)[0]
    order = jnp.argsort(-obj[idxs], stable=True)
    top = idxs[order]
    loc_top = jnp.take(ploc, top, axis=0)
    anc_top = jnp.take(anchor, top, axis=0)
    pad = ((0, _N_PAD - _TOPK), (0, 0))
    boxes_t, keep = _nms(jnp.pad(loc_top, pad).T, jnp.pad(anc_top, pad).T)
    keepb = keep[0, :_TOPK] > 0.5
    kidx = jnp.nonzero(keepb, size=_OUT_K, fill_value=-1)[0]
    valid = kidx >= 0
    gathered = boxes_t.T[jnp.clip(kidx, 0, _TOPK - 1)]
    return jnp.where(valid[:, None], gathered, 0.0)
